# baseline (speedup 1.0000x reference)
import functools

import jax
import jax.numpy as jnp
from jax import lax
from jax.experimental import pallas as pl
from jax.experimental.pallas import tpu as pltpu
from jax.experimental.pallas import tpu_sc as plsc

N_NODES = 100000
IN_DIM = 128
OUT_DIM = 128

_TC_BLK = 1000


def _pack_half_words(a):
    lo = jax.lax.bitcast_convert_type(
        a[:, : OUT_DIM // 2].astype(jnp.bfloat16), jnp.uint16
    ).astype(jnp.uint32)
    hi = jax.lax.bitcast_convert_type(
        a[:, OUT_DIM // 2 :].astype(jnp.bfloat16), jnp.uint16
    ).astype(jnp.uint32)
    return jax.lax.bitcast_convert_type(lo | (hi << 16), jnp.int32)


def _tc_tables_body(x_ref, w_ref, w3_ref, a_ref, b_ref):
    a = jnp.dot(x_ref[...].astype(jnp.bfloat16),
                w_ref[...].astype(jnp.bfloat16),
                preferred_element_type=jnp.float32)
    a_ref[...] = _pack_half_words(a)
    b_ref[...] = _pack_half_words(a * w3_ref[...])


def _make_tables(X, W, w3_row):
    n = X.shape[0]
    grid = (n // _TC_BLK,)
    return pl.pallas_call(
        _tc_tables_body,
        grid=grid,
        in_specs=[
            pl.BlockSpec((_TC_BLK, IN_DIM), lambda i: (i, 0)),
            pl.BlockSpec((IN_DIM, OUT_DIM), lambda i: (0, 0)),
            pl.BlockSpec((1, OUT_DIM), lambda i: (0, 0)),
        ],
        out_specs=[
            pl.BlockSpec((_TC_BLK, OUT_DIM // 2), lambda i: (i, 0)),
            pl.BlockSpec((_TC_BLK, OUT_DIM // 2), lambda i: (i, 0)),
        ],
        out_shape=[
            jax.ShapeDtypeStruct((n, OUT_DIM // 2), jnp.int32),
            jax.ShapeDtypeStruct((n, OUT_DIM // 2), jnp.int32),
        ],
    )(X, W, w3_row)


_C = 128
_CORE0_FRAC = 0.55
_LANES = 16
_KB = OUT_DIM // _LANES


def _make_sc_edge_kernel(e_pad, k0, k1):
    mesh = plsc.VectorSubcoreMesh(core_axis_name="c", subcore_axis_name="s")
    pw0 = k0 * _C
    pw1 = k1 * _C
    pw_max = max(pw0, pw1)

    @functools.partial(
        pl.kernel,
        mesh=mesh,
        compiler_params=pltpu.CompilerParams(needs_layout_passes=False,
                                             use_tc_tiling_on_sc=False),
        out_type=jax.ShapeDtypeStruct((e_pad,), jnp.float32),
        scratch_types=[
            pltpu.VMEM((pw_max,), jnp.int32),
            pltpu.VMEM((pw_max,), jnp.int32),
            pltpu.VMEM((_C, OUT_DIM // 2), jnp.int32),
            pltpu.VMEM((_C, OUT_DIM // 2), jnp.int32),
            pltpu.VMEM((_C, OUT_DIM // 2), jnp.int32),
            pltpu.VMEM((_C, OUT_DIM // 2), jnp.int32),
            pltpu.VMEM((pw_max,), jnp.float32),
            pltpu.SemaphoreType.DMA,
            pltpu.SemaphoreType.DMA,
            pltpu.SemaphoreType.DMA,
            pltpu.SemaphoreType.DMA,
        ],
    )
    def sc_edge(a_hbm, b_hbm, i_hbm, j_hbm, out_hbm,
                idx_i, idx_j, ba0, ba1, bb0, bb1, out_v,
                sa0, sa1, sb0, sb1):
        cidx = lax.axis_index("c")
        sidx = lax.axis_index("s")
        base = sidx * (pw0 + pw1) + cidx * pw0
        my_nchunks = jnp.where(cidx == 0, k0, k1)

        @pl.when(cidx == 0)
        def _():
            pltpu.sync_copy(i_hbm.at[pl.ds(base, pw0)],
                            idx_i.at[pl.ds(0, pw0)])
            pltpu.sync_copy(j_hbm.at[pl.ds(base, pw0)],
                            idx_j.at[pl.ds(0, pw0)])

        @pl.when(cidx == 1)
        def _():
            pltpu.sync_copy(i_hbm.at[pl.ds(base, pw1)],
                            idx_i.at[pl.ds(0, pw1)])
            pltpu.sync_copy(j_hbm.at[pl.ds(base, pw1)],
                            idx_j.at[pl.ds(0, pw1)])

        bufs_a = (ba0, ba1)
        bufs_b = (bb0, bb1)
        sems_a = (sa0, sa1)
        sems_b = (sb0, sb1)

        def start(c, slot):
            pltpu.async_copy(a_hbm.at[idx_i.at[pl.ds(c * _C, _C)]],
                             bufs_a[slot], sems_a[slot])
            pltpu.async_copy(b_hbm.at[idx_j.at[pl.ds(c * _C, _C)]],
                             bufs_b[slot], sems_b[slot])

        def wait(slot):
            pltpu.make_async_copy(a_hbm.at[idx_i.at[pl.ds(0, _C)]],
                                  bufs_a[slot], sems_a[slot]).wait()
            pltpu.make_async_copy(b_hbm.at[idx_j.at[pl.ds(0, _C)]],
                                  bufs_b[slot], sems_b[slot]).wait()

        lane = lax.iota(jnp.int32, _LANES)

        def compute(c, ba, bb):
            out_base = c * _C

            def grp_body(g, carry):
                vec = jnp.zeros((_LANES,), jnp.float32)
                for l in range(_LANES):
                    e = g * _LANES + l
                    acc = None
                    for kb in range(OUT_DIM // 32):
                        a32 = plsc.bitcast(ba[e, pl.ds(kb * 16, 16)],
                                           jnp.bfloat16)
                        b32 = plsc.bitcast(bb[e, pl.ds(kb * 16, 16)],
                                           jnp.bfloat16)
                        a_lo, a_hi = plsc.unpack(
                            a32, format=plsc.PackFormat.INTERLEAVED)
                        b_lo, b_hi = plsc.unpack(
                            b32, format=plsc.PackFormat.INTERLEAVED)
                        t = a_lo * b_lo + a_hi * b_hi
                        acc = t if acc is None else acc + t
                    vec = jnp.where(lane == l, jnp.sum(acc), vec)
                out_v[pl.ds(out_base + g * _LANES, _LANES)] = (
                    1.0 / (1.0 + jnp.exp(-vec)))
                return carry

            lax.fori_loop(0, _C // _LANES, grp_body, 0)

        start(0, 0)

        def outer(h, carry):
            c2 = h * 2
            for b in range(2):
                c = c2 + b
                nxt = c + 1

                @pl.when(nxt < my_nchunks)
                def _():
                    start(nxt, (b + 1) % 2)

                wait(b)
                compute(c, bufs_a[b], bufs_b[b])
            return carry

        lax.fori_loop(0, my_nchunks // 2, outer, 0)

        @pl.when(cidx == 0)
        def _():
            pltpu.sync_copy(out_v.at[pl.ds(0, pw0)],
                            out_hbm.at[pl.ds(base, pw0)])

        @pl.when(cidx == 1)
        def _():
            pltpu.sync_copy(out_v.at[pl.ds(0, pw1)],
                            out_hbm.at[pl.ds(base, pw1)])

    return sc_edge


def kernel(X, train_edges, train_false_edges, W, W3):
    w3_row = W3.reshape(1, OUT_DIM).astype(jnp.float32)
    a_tab, b_tab = _make_tables(X, W, w3_row)

    src = jnp.concatenate([train_edges[:, 0], train_false_edges[:, 0]])
    dst = jnp.concatenate([train_edges[:, 1], train_false_edges[:, 1]])
    src = src.astype(jnp.int32)
    dst = dst.astype(jnp.int32)
    e = src.shape[0]

    info = plsc.get_sparse_core_info()
    ns = info.num_subcores
    k_pair = -(-e // (ns * _C))
    if k_pair % 2:
        k_pair += 1
    k0 = max(2, 2 * round(k_pair * _CORE0_FRAC / 2))
    k1 = k_pair - k0
    e_pad = ns * k_pair * _C

    src_p = jnp.pad(src, (0, e_pad - e))
    dst_p = jnp.pad(dst, (0, e_pad - e))

    sc_fn = _make_sc_edge_kernel(e_pad, k0, k1)
    out_flat = sc_fn(a_tab, b_tab, src_p, dst_p)
    return out_flat[:e].reshape(e, 1)


# speedup vs baseline: 1.1166x; 1.1166x over previous
import functools

import jax
import jax.numpy as jnp
from jax import lax
from jax.experimental import pallas as pl
from jax.experimental.pallas import tpu as pltpu
from jax.experimental.pallas import tpu_sc as plsc

N_NODES = 100000
IN_DIM = 128
OUT_DIM = 128

_TC_BLK = 4000


def _pack_half_words(a):
    lo = jax.lax.bitcast_convert_type(
        a[:, : OUT_DIM // 2].astype(jnp.bfloat16), jnp.uint16
    ).astype(jnp.uint32)
    hi = jax.lax.bitcast_convert_type(
        a[:, OUT_DIM // 2 :].astype(jnp.bfloat16), jnp.uint16
    ).astype(jnp.uint32)
    return jax.lax.bitcast_convert_type(lo | (hi << 16), jnp.int32)


def _tc_tables_body(x_ref, w_ref, w3_ref, a_ref, b_ref):
    a = jnp.dot(x_ref[...].astype(jnp.bfloat16),
                w_ref[...].astype(jnp.bfloat16),
                preferred_element_type=jnp.float32)
    a_ref[...] = _pack_half_words(a)
    b_ref[...] = _pack_half_words(a * w3_ref[...])


def _make_tables(X, W, w3_row):
    n = X.shape[0]
    grid = (n // _TC_BLK,)
    return pl.pallas_call(
        _tc_tables_body,
        grid=grid,
        in_specs=[
            pl.BlockSpec((_TC_BLK, IN_DIM), lambda i: (i, 0)),
            pl.BlockSpec((IN_DIM, OUT_DIM), lambda i: (0, 0)),
            pl.BlockSpec((1, OUT_DIM), lambda i: (0, 0)),
        ],
        out_specs=[
            pl.BlockSpec((_TC_BLK, OUT_DIM // 2), lambda i: (i, 0)),
            pl.BlockSpec((_TC_BLK, OUT_DIM // 2), lambda i: (i, 0)),
        ],
        out_shape=[
            jax.ShapeDtypeStruct((n, OUT_DIM // 2), jnp.int32),
            jax.ShapeDtypeStruct((n, OUT_DIM // 2), jnp.int32),
        ],
    )(X, W, w3_row)


_C = 128
_CORE0_FRAC = 0.55
_LANES = 16
_KB = OUT_DIM // _LANES


def _make_sc_edge_kernel(e_pad, k0, k1):
    mesh = plsc.VectorSubcoreMesh(core_axis_name="c", subcore_axis_name="s")
    pw0 = k0 * _C
    pw1 = k1 * _C
    pw_max = max(pw0, pw1)

    @functools.partial(
        pl.kernel,
        mesh=mesh,
        compiler_params=pltpu.CompilerParams(needs_layout_passes=False,
                                             use_tc_tiling_on_sc=False),
        out_type=jax.ShapeDtypeStruct((e_pad,), jnp.float32),
        scratch_types=[
            pltpu.VMEM((pw_max,), jnp.int32),
            pltpu.VMEM((pw_max,), jnp.int32),
            pltpu.VMEM((_C, OUT_DIM // 2), jnp.int32),
            pltpu.VMEM((_C, OUT_DIM // 2), jnp.int32),
            pltpu.VMEM((_C, OUT_DIM // 2), jnp.int32),
            pltpu.VMEM((_C, OUT_DIM // 2), jnp.int32),
            pltpu.VMEM((pw_max,), jnp.float32),
            pltpu.SemaphoreType.DMA,
            pltpu.SemaphoreType.DMA,
            pltpu.SemaphoreType.DMA,
            pltpu.SemaphoreType.DMA,
        ],
    )
    def sc_edge(a_hbm, b_hbm, i_hbm, j_hbm, out_hbm,
                idx_i, idx_j, ba0, ba1, bb0, bb1, out_v,
                sa0, sa1, sb0, sb1):
        cidx = lax.axis_index("c")
        sidx = lax.axis_index("s")
        base = sidx * (pw0 + pw1) + cidx * pw0
        my_nchunks = jnp.where(cidx == 0, k0, k1)

        @pl.when(cidx == 0)
        def _():
            pltpu.sync_copy(i_hbm.at[pl.ds(base, pw0)],
                            idx_i.at[pl.ds(0, pw0)])
            pltpu.sync_copy(j_hbm.at[pl.ds(base, pw0)],
                            idx_j.at[pl.ds(0, pw0)])

        @pl.when(cidx == 1)
        def _():
            pltpu.sync_copy(i_hbm.at[pl.ds(base, pw1)],
                            idx_i.at[pl.ds(0, pw1)])
            pltpu.sync_copy(j_hbm.at[pl.ds(base, pw1)],
                            idx_j.at[pl.ds(0, pw1)])

        bufs_a = (ba0, ba1)
        bufs_b = (bb0, bb1)
        sems_a = (sa0, sa1)
        sems_b = (sb0, sb1)

        def start(c, slot):
            pltpu.async_copy(a_hbm.at[idx_i.at[pl.ds(c * _C, _C)]],
                             bufs_a[slot], sems_a[slot])
            pltpu.async_copy(b_hbm.at[idx_j.at[pl.ds(c * _C, _C)]],
                             bufs_b[slot], sems_b[slot])

        def wait(slot):
            pltpu.make_async_copy(a_hbm.at[idx_i.at[pl.ds(0, _C)]],
                                  bufs_a[slot], sems_a[slot]).wait()
            pltpu.make_async_copy(b_hbm.at[idx_j.at[pl.ds(0, _C)]],
                                  bufs_b[slot], sems_b[slot]).wait()

        lane = lax.iota(jnp.int32, _LANES)

        def compute(c, ba, bb):
            out_base = c * _C

            def grp_body(g, carry):
                vec = jnp.zeros((_LANES,), jnp.float32)
                for l in range(_LANES):
                    e = g * _LANES + l
                    acc = None
                    for kb in range(OUT_DIM // 32):
                        a32 = plsc.bitcast(ba[e, pl.ds(kb * 16, 16)],
                                           jnp.bfloat16)
                        b32 = plsc.bitcast(bb[e, pl.ds(kb * 16, 16)],
                                           jnp.bfloat16)
                        a_lo, a_hi = plsc.unpack(
                            a32, format=plsc.PackFormat.INTERLEAVED)
                        b_lo, b_hi = plsc.unpack(
                            b32, format=plsc.PackFormat.INTERLEAVED)
                        t = a_lo * b_lo + a_hi * b_hi
                        acc = t if acc is None else acc + t
                    vec = jnp.where(lane == l, jnp.sum(acc), vec)
                out_v[pl.ds(out_base + g * _LANES, _LANES)] = (
                    1.0 / (1.0 + jnp.exp(-vec)))
                return carry

            lax.fori_loop(0, _C // _LANES, grp_body, 0)

        start(0, 0)

        def outer(h, carry):
            c2 = h * 2
            for b in range(2):
                c = c2 + b
                nxt = c + 1

                @pl.when(nxt < my_nchunks)
                def _():
                    start(nxt, (b + 1) % 2)

                wait(b)
                compute(c, bufs_a[b], bufs_b[b])
            return carry

        lax.fori_loop(0, my_nchunks // 2, outer, 0)

        @pl.when(cidx == 0)
        def _():
            pltpu.sync_copy(out_v.at[pl.ds(0, pw0)],
                            out_hbm.at[pl.ds(base, pw0)])

        @pl.when(cidx == 1)
        def _():
            pltpu.sync_copy(out_v.at[pl.ds(0, pw1)],
                            out_hbm.at[pl.ds(base, pw1)])

    return sc_edge


def kernel(X, train_edges, train_false_edges, W, W3):
    w3_row = W3.reshape(1, OUT_DIM).astype(jnp.float32)
    a_tab, b_tab = _make_tables(X, W, w3_row)

    src = jnp.concatenate([train_edges[:, 0], train_false_edges[:, 0]])
    dst = jnp.concatenate([train_edges[:, 1], train_false_edges[:, 1]])
    src = src.astype(jnp.int32)
    dst = dst.astype(jnp.int32)
    e = src.shape[0]

    info = plsc.get_sparse_core_info()
    ns = info.num_subcores
    k_pair = -(-e // (ns * _C))
    if k_pair % 2:
        k_pair += 1
    k0 = max(2, 2 * round(k_pair * _CORE0_FRAC / 2))
    k1 = k_pair - k0
    e_pad = ns * k_pair * _C

    src_p = jnp.pad(src, (0, e_pad - e))
    dst_p = jnp.pad(dst, (0, e_pad - e))

    sc_fn = _make_sc_edge_kernel(e_pad, k0, k1)
    out_flat = sc_fn(a_tab, b_tab, src_p, dst_p)
    return out_flat[:e].reshape(e, 1)


# speedup vs baseline: 1.1354x; 1.0169x over previous
import functools

import jax
import jax.numpy as jnp
from jax import lax
from jax.experimental import pallas as pl
from jax.experimental.pallas import tpu as pltpu
from jax.experimental.pallas import tpu_sc as plsc

N_NODES = 100000
IN_DIM = 128
OUT_DIM = 128

_TC_BLK = 10000


def _pack_half_words(a):
    lo = jax.lax.bitcast_convert_type(
        a[:, : OUT_DIM // 2].astype(jnp.bfloat16), jnp.uint16
    ).astype(jnp.uint32)
    hi = jax.lax.bitcast_convert_type(
        a[:, OUT_DIM // 2 :].astype(jnp.bfloat16), jnp.uint16
    ).astype(jnp.uint32)
    return jax.lax.bitcast_convert_type(lo | (hi << 16), jnp.int32)


def _tc_tables_body(x_ref, w_ref, w3_ref, a_ref, b_ref):
    a = jnp.dot(x_ref[...].astype(jnp.bfloat16),
                w_ref[...].astype(jnp.bfloat16),
                preferred_element_type=jnp.float32)
    a_ref[...] = _pack_half_words(a)
    b_ref[...] = _pack_half_words(a * w3_ref[...])


def _make_tables(X, W, w3_row):
    n = X.shape[0]
    grid = (n // _TC_BLK,)
    return pl.pallas_call(
        _tc_tables_body,
        grid=grid,
        in_specs=[
            pl.BlockSpec((_TC_BLK, IN_DIM), lambda i: (i, 0)),
            pl.BlockSpec((IN_DIM, OUT_DIM), lambda i: (0, 0)),
            pl.BlockSpec((1, OUT_DIM), lambda i: (0, 0)),
        ],
        out_specs=[
            pl.BlockSpec((_TC_BLK, OUT_DIM // 2), lambda i: (i, 0)),
            pl.BlockSpec((_TC_BLK, OUT_DIM // 2), lambda i: (i, 0)),
        ],
        out_shape=[
            jax.ShapeDtypeStruct((n, OUT_DIM // 2), jnp.int32),
            jax.ShapeDtypeStruct((n, OUT_DIM // 2), jnp.int32),
        ],
    )(X, W, w3_row)


_C = 128
_CORE0_FRAC = 0.55
_LANES = 16
_KB = OUT_DIM // _LANES


def _make_sc_edge_kernel(e_pad, k0, k1):
    mesh = plsc.VectorSubcoreMesh(core_axis_name="c", subcore_axis_name="s")
    pw0 = k0 * _C
    pw1 = k1 * _C
    pw_max = max(pw0, pw1)

    @functools.partial(
        pl.kernel,
        mesh=mesh,
        compiler_params=pltpu.CompilerParams(needs_layout_passes=False,
                                             use_tc_tiling_on_sc=False),
        out_type=jax.ShapeDtypeStruct((e_pad,), jnp.float32),
        scratch_types=[
            pltpu.VMEM((pw_max,), jnp.int32),
            pltpu.VMEM((pw_max,), jnp.int32),
            pltpu.VMEM((_C, OUT_DIM // 2), jnp.int32),
            pltpu.VMEM((_C, OUT_DIM // 2), jnp.int32),
            pltpu.VMEM((_C, OUT_DIM // 2), jnp.int32),
            pltpu.VMEM((_C, OUT_DIM // 2), jnp.int32),
            pltpu.VMEM((pw_max,), jnp.float32),
            pltpu.SemaphoreType.DMA,
            pltpu.SemaphoreType.DMA,
            pltpu.SemaphoreType.DMA,
            pltpu.SemaphoreType.DMA,
        ],
    )
    def sc_edge(a_hbm, b_hbm, i_hbm, j_hbm, out_hbm,
                idx_i, idx_j, ba0, ba1, bb0, bb1, out_v,
                sa0, sa1, sb0, sb1):
        cidx = lax.axis_index("c")
        sidx = lax.axis_index("s")
        base = sidx * (pw0 + pw1) + cidx * pw0
        my_nchunks = jnp.where(cidx == 0, k0, k1)

        @pl.when(cidx == 0)
        def _():
            pltpu.sync_copy(i_hbm.at[pl.ds(base, pw0)],
                            idx_i.at[pl.ds(0, pw0)])
            pltpu.sync_copy(j_hbm.at[pl.ds(base, pw0)],
                            idx_j.at[pl.ds(0, pw0)])

        @pl.when(cidx == 1)
        def _():
            pltpu.sync_copy(i_hbm.at[pl.ds(base, pw1)],
                            idx_i.at[pl.ds(0, pw1)])
            pltpu.sync_copy(j_hbm.at[pl.ds(base, pw1)],
                            idx_j.at[pl.ds(0, pw1)])

        bufs_a = (ba0, ba1)
        bufs_b = (bb0, bb1)
        sems_a = (sa0, sa1)
        sems_b = (sb0, sb1)

        def start(c, slot):
            pltpu.async_copy(a_hbm.at[idx_i.at[pl.ds(c * _C, _C)]],
                             bufs_a[slot], sems_a[slot])
            pltpu.async_copy(b_hbm.at[idx_j.at[pl.ds(c * _C, _C)]],
                             bufs_b[slot], sems_b[slot])

        def wait(slot):
            pltpu.make_async_copy(a_hbm.at[idx_i.at[pl.ds(0, _C)]],
                                  bufs_a[slot], sems_a[slot]).wait()
            pltpu.make_async_copy(b_hbm.at[idx_j.at[pl.ds(0, _C)]],
                                  bufs_b[slot], sems_b[slot]).wait()

        lane = lax.iota(jnp.int32, _LANES)

        def compute(c, ba, bb):
            out_base = c * _C

            def grp_body(g, carry):
                vec = jnp.zeros((_LANES,), jnp.float32)
                for l in range(_LANES):
                    e = g * _LANES + l
                    acc = None
                    for kb in range(OUT_DIM // 32):
                        a32 = plsc.bitcast(ba[e, pl.ds(kb * 16, 16)],
                                           jnp.bfloat16)
                        b32 = plsc.bitcast(bb[e, pl.ds(kb * 16, 16)],
                                           jnp.bfloat16)
                        a_lo, a_hi = plsc.unpack(
                            a32, format=plsc.PackFormat.INTERLEAVED)
                        b_lo, b_hi = plsc.unpack(
                            b32, format=plsc.PackFormat.INTERLEAVED)
                        t = a_lo * b_lo + a_hi * b_hi
                        acc = t if acc is None else acc + t
                    vec = jnp.where(lane == l, jnp.sum(acc), vec)
                out_v[pl.ds(out_base + g * _LANES, _LANES)] = (
                    1.0 / (1.0 + jnp.exp(-vec)))
                return carry

            lax.fori_loop(0, _C // _LANES, grp_body, 0)

        start(0, 0)

        def outer(h, carry):
            c2 = h * 2
            for b in range(2):
                c = c2 + b
                nxt = c + 1

                @pl.when(nxt < my_nchunks)
                def _():
                    start(nxt, (b + 1) % 2)

                wait(b)
                compute(c, bufs_a[b], bufs_b[b])
            return carry

        lax.fori_loop(0, my_nchunks // 2, outer, 0)

        @pl.when(cidx == 0)
        def _():
            pltpu.sync_copy(out_v.at[pl.ds(0, pw0)],
                            out_hbm.at[pl.ds(base, pw0)])

        @pl.when(cidx == 1)
        def _():
            pltpu.sync_copy(out_v.at[pl.ds(0, pw1)],
                            out_hbm.at[pl.ds(base, pw1)])

    return sc_edge


def kernel(X, train_edges, train_false_edges, W, W3):
    w3_row = W3.reshape(1, OUT_DIM).astype(jnp.float32)
    a_tab, b_tab = _make_tables(X, W, w3_row)

    src = jnp.concatenate([train_edges[:, 0], train_false_edges[:, 0]])
    dst = jnp.concatenate([train_edges[:, 1], train_false_edges[:, 1]])
    src = src.astype(jnp.int32)
    dst = dst.astype(jnp.int32)
    e = src.shape[0]

    info = plsc.get_sparse_core_info()
    ns = info.num_subcores
    k_pair = -(-e // (ns * _C))
    if k_pair % 2:
        k_pair += 1
    k0 = max(2, 2 * round(k_pair * _CORE0_FRAC / 2))
    k1 = k_pair - k0
    e_pad = ns * k_pair * _C

    src_p = jnp.pad(src, (0, e_pad - e))
    dst_p = jnp.pad(dst, (0, e_pad - e))

    sc_fn = _make_sc_edge_kernel(e_pad, k0, k1)
    out_flat = sc_fn(a_tab, b_tab, src_p, dst_p)
    return out_flat[:e].reshape(e, 1)
